# baseline (speedup 1.0000x reference)
import functools
import math

import jax
import jax.numpy as jnp
from jax import lax
from jax.experimental import pallas as pl
from jax.experimental.pallas import tpu as pltpu
from jax.experimental.pallas import tpu_sc as plsc

HID = 64
SCALE = math.sqrt(HID)
PADW = 128

NC = 2
NS = 16
NW = NC * NS
LANES = 16
BBLK = 128


def _emb_body(seq, n_batch, idsT_hbm, table_hbm, out_hbm,
              idx_v, g0, g1, s0, s1, isem, gs0, gs1, ws0, ws1):
    wid = lax.axis_index("s") * NC + lax.axis_index("c")
    b0 = wid * BBLK
    gbuf = (g0, g1)
    sbuf = (s0, s1)
    gsem = (gs0, gs1)
    wsem = (ws0, ws1)

    for t in range(seq // 8):
        pltpu.async_copy(idsT_hbm.at[pl.ds(t * 8, 8), pl.ds(b0, BBLK)],
                         idx_v.at[pl.ds(t * 8, 8)], isem)
    for t in range(seq // 8):
        pltpu.make_async_copy(idsT_hbm.at[pl.ds(0, 8), pl.ds(0, BBLK)],
                              idx_v.at[pl.ds(0, 8)], isem).wait()

    def start_gather(s, b):
        pltpu.async_copy(table_hbm.at[idx_v.at[s]], gbuf[b], gsem[b])

    def wait_gather(b):
        pltpu.make_async_copy(table_hbm.at[idx_v.at[0]], gbuf[b], gsem[b]).wait()

    def start_write(s, b):
        pltpu.async_copy(sbuf[b], out_hbm.at[s, :, pl.ds(b0, BBLK)], wsem[b])

    def wait_write(b):
        pltpu.make_async_copy(sbuf[b], out_hbm.at[0, :, pl.ds(0, BBLK)],
                              wsem[b]).wait()

    iota = lax.iota(jnp.int32, LANES)
    rows = [iota + (blk * LANES) for blk in range(BBLK // LANES)]

    def scale_t(b):
        def col_blk(h, carry):
            cols = jnp.full((LANES,), h, jnp.int32)
            for blk in range(BBLK // LANES):
                v = plsc.load_gather(gbuf[b], [rows[blk], cols])
                sbuf[b][h, pl.ds(blk * LANES, LANES)] = v * SCALE
            return carry

        lax.fori_loop(0, HID, col_blk, 0)

    start_gather(0, 0)
    start_gather(1, 1)

    for s in (0, 1):
        b = s
        wait_gather(b)
        scale_t(b)
        start_gather(s + 2, b)
        start_write(s, b)

    @pl.loop(2, seq - 2, step=2)
    def _(s0_):
        for b in range(2):
            s = s0_ + b
            wait_gather(b)
            wait_write(b)
            scale_t(b)
            start_gather(s + 2, b)
            start_write(s, b)

    for b in range(2):
        s = seq - 2 + b
        wait_gather(b)
        wait_write(b)
        scale_t(b)
        start_write(s, b)
    for b in range(2):
        wait_write(b)


def _make_emb(seq, n_batch, vocab):
    assert n_batch % (NW * BBLK) == 0 or n_batch == NW * BBLK
    mesh = plsc.VectorSubcoreMesh(core_axis_name="c", subcore_axis_name="s")
    return pl.kernel(
        functools.partial(_emb_body, seq, n_batch),
        out_type=jax.ShapeDtypeStruct((seq, HID, n_batch), jnp.float32),
        mesh=mesh,
        scratch_types=[
            pltpu.VMEM((seq, BBLK), jnp.int32),
            pltpu.VMEM((BBLK, PADW), jnp.float32),
            pltpu.VMEM((BBLK, PADW), jnp.float32),
            pltpu.VMEM((HID, BBLK), jnp.float32),
            pltpu.VMEM((HID, BBLK), jnp.float32),
            pltpu.SemaphoreType.DMA,
            pltpu.SemaphoreType.DMA,
            pltpu.SemaphoreType.DMA,
            pltpu.SemaphoreType.DMA,
            pltpu.SemaphoreType.DMA,
        ],
        compiler_params=pltpu.CompilerParams(use_tc_tiling_on_sc=True,
                                             needs_layout_passes=False),
    )


def kernel(input_ids, table):
    n_batch, seq = input_ids.shape
    vocab = table.shape[0]
    idsT = input_ids.T.astype(jnp.int32)
    tpad = jnp.pad(table, ((0, 0), (0, PADW - HID)))
    out_phys = _make_emb(seq, n_batch, vocab)(idsT, tpad)
    return out_phys.transpose(2, 0, 1)


# speedup vs baseline: 1.5044x; 1.5044x over previous
import functools
import math

import jax
import jax.numpy as jnp
from jax import lax
from jax.experimental import pallas as pl
from jax.experimental.pallas import tpu as pltpu
from jax.experimental.pallas import tpu_sc as plsc

HID = 64
SCALE = math.sqrt(HID)
PADW = 128

NC = 2
NS = 16
NW = NC * NS
LANES = 16
BBLK = 128


def _emb_body(seq, n_batch, idsT_hbm, table_hbm, out_hbm,
              idx_v, hi_v, g0, g1, s0, s1, isem, gs0, gs1, ws0, ws1):
    wid = lax.axis_index("s") * NC + lax.axis_index("c")
    b0 = wid * BBLK
    gbuf = (g0, g1)
    sbuf = (s0, s1)
    gsem = (gs0, gs1)
    wsem = (ws0, ws1)

    for t in range(seq // 8):
        pltpu.async_copy(idsT_hbm.at[pl.ds(t * 8, 8), pl.ds(b0, BBLK)],
                         idx_v.at[pl.ds(t * 8, 8)], isem)
    for t in range(seq // 8):
        pltpu.make_async_copy(idsT_hbm.at[pl.ds(0, 8), pl.ds(0, BBLK)],
                              idx_v.at[pl.ds(0, 8)], isem).wait()

    def split_ids(i, carry):
        s = i // (BBLK // LANES)
        off = (i % (BBLK // LANES)) * LANES
        v = idx_v[s, pl.ds(off, LANES)]
        hi_v[s, pl.ds(off, LANES)] = (v & 1) << 6
        idx_v[s, pl.ds(off, LANES)] = lax.shift_right_logical(v, 1)
        return carry

    lax.fori_loop(0, seq * (BBLK // LANES), split_ids, 0)

    def start_gather(s, b):
        pltpu.async_copy(table_hbm.at[idx_v.at[s]], gbuf[b], gsem[b])

    def wait_gather(b):
        pltpu.make_async_copy(table_hbm.at[idx_v.at[0]], gbuf[b], gsem[b]).wait()

    def start_write(s, b):
        pltpu.async_copy(sbuf[b], out_hbm.at[s, :, pl.ds(b0, BBLK)], wsem[b])

    def wait_write(b):
        pltpu.make_async_copy(sbuf[b], out_hbm.at[0, :, pl.ds(0, BBLK)],
                              wsem[b]).wait()

    iota = lax.iota(jnp.int32, LANES)
    rot = [(iota + k) & (LANES - 1) for k in range(LANES)]

    def transpose_scale(s, b):
        def r_blk(rb, carry):
            r0 = rb * LANES
            rows = iota + r0
            par = hi_v[s, pl.ds(r0, LANES)]
            for hbase in range(0, HID, LANES):
                colbase = par + hbase
                for k in range(LANES):
                    v = plsc.load_gather(gbuf[b], [rows, colbase + rot[k]])
                    plsc.store_scatter(sbuf[b], [rot[k] + hbase, rows],
                                       v * SCALE)
            return carry

        lax.fori_loop(0, BBLK // LANES, r_blk, 0)

    start_gather(0, 0)
    start_gather(1, 1)

    for s in (0, 1):
        b = s
        wait_gather(b)
        transpose_scale(s, b)
        start_gather(s + 2, b)
        start_write(s, b)

    @pl.loop(2, seq - 2, step=2)
    def _(s0_):
        for b in range(2):
            s = s0_ + b
            wait_gather(b)
            wait_write(b)
            transpose_scale(s, b)
            start_gather(s + 2, b)
            start_write(s, b)

    for b in range(2):
        s = seq - 2 + b
        wait_gather(b)
        wait_write(b)
        transpose_scale(s, b)
        start_write(s, b)
    for b in range(2):
        wait_write(b)


def _make_emb(seq, n_batch):
    assert n_batch == NW * BBLK
    mesh = plsc.VectorSubcoreMesh(core_axis_name="c", subcore_axis_name="s")
    return pl.kernel(
        functools.partial(_emb_body, seq, n_batch),
        out_type=jax.ShapeDtypeStruct((seq, HID, n_batch), jnp.float32),
        mesh=mesh,
        scratch_types=[
            pltpu.VMEM((seq, BBLK), jnp.int32),
            pltpu.VMEM((seq, BBLK), jnp.int32),
            pltpu.VMEM((BBLK, PADW), jnp.float32),
            pltpu.VMEM((BBLK, PADW), jnp.float32),
            pltpu.VMEM((HID, BBLK), jnp.float32),
            pltpu.VMEM((HID, BBLK), jnp.float32),
            pltpu.SemaphoreType.DMA,
            pltpu.SemaphoreType.DMA,
            pltpu.SemaphoreType.DMA,
            pltpu.SemaphoreType.DMA,
            pltpu.SemaphoreType.DMA,
        ],
        compiler_params=pltpu.CompilerParams(use_tc_tiling_on_sc=True,
                                             needs_layout_passes=False),
    )


def kernel(input_ids, table):
    n_batch, seq = input_ids.shape
    vocab = table.shape[0]
    idsT = input_ids.T.astype(jnp.int32)
    tview = table.reshape(vocab // 2, PADW)
    out_phys = _make_emb(seq, n_batch)(idsT, tview)
    return out_phys.transpose(2, 0, 1)
